# Initial kernel scaffold; baseline (speedup 1.0000x reference)
import jax
import jax.numpy as jnp
from jax.experimental import pallas as pl


def kernel(artist_name_can, track_name_can, album_name_can, artist_uri_can, track_uri_can, album_uri_can, duration_ms_can, track_pop_can, artist_pop_can, artist_followers_can, artist_genres_can, t_artist_name, t_track_name, t_album_name, t_artist_uri, t_track_uri, t_album_uri, t_duration, t_track_pop, t_artist_pop, t_followers, t_genres, W1, b1, W2, b2, W3, b3):
    raise NotImplementedError("write your pallas kernel here")


# speedup vs baseline: 1.6423x; 1.6423x over previous
import functools

import jax
import jax.numpy as jnp
from jax import lax
from jax.experimental import pallas as pl
from jax.experimental.pallas import tpu as pltpu
from jax.experimental.pallas import tpu_sc as plsc

B = 4096
D = 512
NF = 11
DIN = NF * D
NC, NS = 2, 16
NW = NC * NS
SPW = B // NW

L_TXT = 20
L_GEN = 8
G_TXT = 4
G_GEN = 8
LANES = 16

SLOT_ARTIST_NAME = 0
SLOT_TRACK_NAME = 1
SLOT_ALBUM_NAME = 2
SLOT_ARTIST_URI = 3
SLOT_TRACK_URI = 4
SLOT_ALBUM_URI = 5
SLOT_DURATION = 6
SLOT_TRACK_POP = 7
SLOT_ARTIST_POP = 8
SLOT_FOLLOWERS = 9
SLOT_GENRES = 10

_MESH = plsc.VectorSubcoreMesh(
    core_axis_name="c", subcore_axis_name="s", num_cores=NC, num_subcores=NS
)


def _sc_body(
    an_i, au_i, tu_i, bu_i, du_i, tp_i, ap_i, fo_i,
    tn_i, bn_i, ge_i,
    t_an, t_tn, t_bn, t_au, t_tu, t_bu, t_du, t_tp, t_ap, t_fo, t_ge,
    out,
    idx_s, idx_m, buf, rowbuf, accst, sem,
):
    wid = lax.axis_index("s") * NC + lax.axis_index("c")
    base = wid * SPW

    singles = (
        (an_i, t_an, SLOT_ARTIST_NAME),
        (au_i, t_au, SLOT_ARTIST_URI),
        (tu_i, t_tu, SLOT_TRACK_URI),
        (bu_i, t_bu, SLOT_ALBUM_URI),
        (du_i, t_du, SLOT_DURATION),
        (tp_i, t_tp, SLOT_TRACK_POP),
        (ap_i, t_ap, SLOT_ARTIST_POP),
        (fo_i, t_fo, SLOT_FOLLOWERS),
    )
    for idx_hbm, tab, slot in singles:
        pltpu.sync_copy(idx_hbm.at[pl.ds(base, SPW)], idx_s)
        pltpu.async_copy(tab.at[idx_s], buf, sem).wait()
        pltpu.sync_copy(buf, out.at[pl.ds(base, SPW), pl.ds(slot * D, D)])

    def mean_field(idxf_hbm, tab, slot, L, G):
        n_rows = G * L
        inv = jnp.float32(1.0 / L)

        def chunk_body(c, carry):
            flat0 = base * L + c * n_rows
            pltpu.sync_copy(
                idxf_hbm.at[pl.ds(flat0, n_rows)], idx_m.at[pl.ds(0, n_rows)]
            )
            pltpu.async_copy(
                tab.at[idx_m.at[pl.ds(0, n_rows)]],
                rowbuf.at[pl.ds(0, n_rows)],
                sem,
            ).wait()
            def col_body(kk, carry2):
                col = pl.ds(pl.multiple_of(kk * LANES, LANES), LANES)
                for g in range(G):
                    parts = [rowbuf[g * L + j, col] for j in range(L)]
                    while len(parts) > 1:
                        parts = [
                            parts[i] + parts[i + 1]
                            if i + 1 < len(parts) else parts[i]
                            for i in range(0, len(parts), 2)
                        ]
                    accst[g, col] = parts[0] * inv
                return carry2

            lax.fori_loop(0, D // LANES, col_body, 0)
            pltpu.sync_copy(
                accst.at[pl.ds(0, G)],
                out.at[pl.ds(base + c * G, G), pl.ds(slot * D, D)],
            )
            return carry

        lax.fori_loop(0, SPW // G, chunk_body, 0)

    mean_field(tn_i, t_tn, SLOT_TRACK_NAME, L_TXT, G_TXT)
    mean_field(bn_i, t_bn, SLOT_ALBUM_NAME, L_TXT, G_TXT)
    mean_field(ge_i, t_ge, SLOT_GENRES, L_GEN, G_GEN)


_sc_gather = functools.partial(
    pl.kernel,
    out_type=jax.ShapeDtypeStruct((B, DIN), jnp.float32),
    mesh=_MESH,
    scratch_types=[
        pltpu.VMEM((SPW,), jnp.int32),
        pltpu.VMEM((G_TXT * L_TXT,), jnp.int32),
        pltpu.VMEM((SPW, D), jnp.float32),
        pltpu.VMEM((G_TXT * L_TXT, D), jnp.float32),
        pltpu.VMEM((G_GEN, D), jnp.float32),
        pltpu.SemaphoreType.DMA,
    ],
)(_sc_body)


def _mlp_body(x_ref, w1_ref, b1_ref, w2_ref, b2_ref, w3_ref, b3_ref,
              out_ref, acc_ref):
    k = pl.program_id(1)
    part = jnp.dot(x_ref[...], w1_ref[...], preferred_element_type=jnp.float32)

    @pl.when(k == 0)
    def _():
        acc_ref[...] = part

    @pl.when(k > 0)
    def _():
        acc_ref[...] += part

    @pl.when(k == NF - 1)
    def _():
        h1 = jnp.maximum(acc_ref[...] + b1_ref[...], 0.0)
        h2 = jnp.maximum(
            jnp.dot(h1, w2_ref[...], preferred_element_type=jnp.float32)
            + b2_ref[...], 0.0)
        out_ref[...] = (
            jnp.dot(h2, w3_ref[...], preferred_element_type=jnp.float32)
            + b3_ref[...])


BM = 256


def _mlp(x, W1, b1, W2, b2, W3, b3):
    return pl.pallas_call(
        _mlp_body,
        grid=(B // BM, NF),
        in_specs=[
            pl.BlockSpec((BM, D), lambda i, k: (i, k)),
            pl.BlockSpec((D, 512), lambda i, k: (k, 0)),
            pl.BlockSpec((1, 512), lambda i, k: (0, 0)),
            pl.BlockSpec((512, 256), lambda i, k: (0, 0)),
            pl.BlockSpec((1, 256), lambda i, k: (0, 0)),
            pl.BlockSpec((256, 128), lambda i, k: (0, 0)),
            pl.BlockSpec((1, 128), lambda i, k: (0, 0)),
        ],
        out_specs=pl.BlockSpec((BM, 128), lambda i, k: (i, 0)),
        out_shape=jax.ShapeDtypeStruct((B, 128), jnp.float32),
        scratch_shapes=[pltpu.VMEM((BM, 512), jnp.float32)],
        compiler_params=pltpu.CompilerParams(
            dimension_semantics=("parallel", "arbitrary"),
        ),
    )(x, W1, b1.reshape(1, 512), W2, b2.reshape(1, 256),
      W3, b3.reshape(1, 128))


def kernel(artist_name_can, track_name_can, album_name_can, artist_uri_can,
           track_uri_can, album_uri_can, duration_ms_can, track_pop_can,
           artist_pop_can, artist_followers_can, artist_genres_can,
           t_artist_name, t_track_name, t_album_name, t_artist_uri,
           t_track_uri, t_album_uri, t_duration, t_track_pop, t_artist_pop,
           t_followers, t_genres, W1, b1, W2, b2, W3, b3):
    i32 = jnp.int32
    embs = _sc_gather(
        artist_name_can.astype(i32),
        artist_uri_can.astype(i32),
        track_uri_can.astype(i32),
        album_uri_can.astype(i32),
        duration_ms_can.astype(i32),
        track_pop_can.astype(i32),
        artist_pop_can.astype(i32),
        artist_followers_can.astype(i32),
        track_name_can.astype(i32).reshape(-1),
        album_name_can.astype(i32).reshape(-1),
        artist_genres_can.astype(i32).reshape(-1),
        t_artist_name, t_track_name, t_album_name, t_artist_uri, t_track_uri,
        t_album_uri, t_duration, t_track_pop, t_artist_pop, t_followers,
        t_genres,
    )
    return _mlp(embs, W1, b1, W2, b2, W3, b3)
